# Initial kernel scaffold; baseline (speedup 1.0000x reference)
import jax
import jax.numpy as jnp
from jax.experimental import pallas as pl


def kernel(positions, amplitudes, hermite_basis):
    raise NotImplementedError("write your pallas kernel here")


# speedup vs baseline: 49.5766x; 49.5766x over previous
import functools

import jax
import jax.numpy as jnp
from jax import lax
from jax.experimental import pallas as pl
from jax.experimental.pallas import tpu as pltpu
from jax.experimental.pallas import tpu_sc as plsc

MAXN = 8
RES = 256
NC = 2
NS = 16
NW = NC * NS
LANES = 16
CHUNK = 32


def _comb_body(amp_ref, basis_ref, comb_ref):
    comb_ref[...] = jnp.dot(
        amp_ref[...], basis_ref[...], preferred_element_type=jnp.float32
    )


def _make_comb(amplitudes, basis):
    batch = amplitudes.shape[0]
    blk = 2048
    return pl.pallas_call(
        _comb_body,
        grid=(batch // blk,),
        in_specs=[
            pl.BlockSpec((blk, MAXN), lambda i: (i, 0)),
            pl.BlockSpec((MAXN, RES), lambda i: (0, 0)),
        ],
        out_specs=pl.BlockSpec((blk, RES), lambda i: (i, 0)),
        out_shape=jax.ShapeDtypeStruct((batch, RES), jnp.float32),
    )(amplitudes, basis)


def _sc_gather(pos_flat, comb_flat, batch, seq_len):
    rows_per_w = batch // NW
    n_chunks = rows_per_w // CHUNK
    chunk_elems = CHUNK * seq_len
    n_vregs = chunk_elems // LANES
    mesh = plsc.VectorSubcoreMesh(core_axis_name="c", subcore_axis_name="s")

    @functools.partial(
        pl.kernel,
        out_type=jax.ShapeDtypeStruct((batch * seq_len,), jnp.float32),
        mesh=mesh,
        scratch_types=[
            pltpu.VMEM((chunk_elems,), jnp.float32),
            pltpu.VMEM((CHUNK * RES,), jnp.float32),
            pltpu.VMEM((chunk_elems,), jnp.float32),
        ],
        compiler_params=pltpu.CompilerParams(needs_layout_passes=False),
    )
    def k(pos_hbm, comb_hbm, out_hbm, pos_v, comb_v, out_v):
        wid = lax.axis_index("s") * NC + lax.axis_index("c")
        lane = lax.iota(jnp.int32, LANES)

        def chunk_body(kk, carry):
            row0 = wid * rows_per_w + kk * CHUNK
            e0 = row0 * seq_len
            pltpu.sync_copy(pos_hbm.at[pl.ds(e0, chunk_elems)], pos_v)
            pltpu.sync_copy(comb_hbm.at[pl.ds(row0 * RES, CHUNK * RES)], comb_v)

            def vec_body(v, c2):
                off = v * LANES
                eloc = off + lane
                r = ((eloc.astype(jnp.float32) + 0.5) * (1.0 / seq_len)).astype(
                    jnp.int32
                )
                p = pos_v[pl.ds(off, LANES)]
                t = (p + 1.0) * 0.5 * 255.0
                idx = jnp.clip(t.astype(jnp.int32), 0, 255)
                g = plsc.load_gather(comb_v, [r * RES + idx])
                out_v[pl.ds(off, LANES)] = g
                return c2

            lax.fori_loop(0, n_vregs, vec_body, 0, unroll=4)
            pltpu.sync_copy(out_v, out_hbm.at[pl.ds(e0, chunk_elems)])
            return carry

        lax.fori_loop(0, n_chunks, chunk_body, 0)

    return k(pos_flat, comb_flat)


def kernel(positions, amplitudes, hermite_basis):
    batch, seq_len = positions.shape
    comb = _make_comb(amplitudes, hermite_basis)
    out = _sc_gather(
        positions.reshape(-1), comb.reshape(-1), batch, seq_len
    )
    return out.reshape(batch, seq_len)


# speedup vs baseline: 107.7705x; 2.1738x over previous
import functools

import jax
import jax.numpy as jnp
from jax import lax
from jax.experimental import pallas as pl
from jax.experimental.pallas import tpu as pltpu
from jax.experimental.pallas import tpu_sc as plsc

MAXN = 8
RES = 256
NC = 2
NS = 16
NW = NC * NS
LANES = 16
CHUNK = 32


def _comb_body(amp_ref, basis_ref, comb_ref):
    comb_ref[...] = jnp.dot(
        amp_ref[...], basis_ref[...], preferred_element_type=jnp.float32
    )


def _make_comb(amplitudes, basis):
    batch = amplitudes.shape[0]
    blk = 2048
    return pl.pallas_call(
        _comb_body,
        grid=(batch // blk,),
        in_specs=[
            pl.BlockSpec((blk, MAXN), lambda i: (i, 0)),
            pl.BlockSpec((MAXN, RES), lambda i: (0, 0)),
        ],
        out_specs=pl.BlockSpec((blk, RES), lambda i: (i, 0)),
        out_shape=jax.ShapeDtypeStruct((batch, RES), jnp.float32),
    )(amplitudes, basis)


def _sc_gather(positions, comb, batch, seq_len):
    rows_per_w = batch // NW
    n_chunks = rows_per_w // CHUNK
    offs = [o * LANES for o in range(seq_len // LANES)]
    if seq_len % LANES:
        offs.append(seq_len - LANES)
    mesh = plsc.VectorSubcoreMesh(core_axis_name="c", subcore_axis_name="s")

    @functools.partial(
        pl.kernel,
        out_type=jax.ShapeDtypeStruct((batch, seq_len), jnp.float32),
        mesh=mesh,
        scratch_types=[
            pltpu.VMEM((CHUNK, seq_len), jnp.float32),
            pltpu.VMEM((CHUNK, RES), jnp.float32),
            pltpu.VMEM((CHUNK, seq_len), jnp.float32),
        ],
        compiler_params=pltpu.CompilerParams(needs_layout_passes=False),
    )
    def k(pos_hbm, comb_hbm, out_hbm, pos_v, comb_v, out_v):
        wid = lax.axis_index("s") * NC + lax.axis_index("c")

        def chunk_body(kk, carry):
            row0 = wid * rows_per_w + kk * CHUNK
            pltpu.sync_copy(pos_hbm.at[pl.ds(row0, CHUNK), :], pos_v)
            pltpu.sync_copy(comb_hbm.at[pl.ds(row0, CHUNK), :], comb_v)

            @plsc.parallel_loop(0, CHUNK, unroll=2)
            def row_body(r):
                rr = jnp.full((LANES,), r, dtype=jnp.int32)
                for off in offs:
                    p = pos_v[r, pl.ds(off, LANES)]
                    t = (p + 1.0) * 0.5 * 255.0
                    idx = jnp.clip(t.astype(jnp.int32), 0, 255)
                    out_v[r, pl.ds(off, LANES)] = plsc.load_gather(
                        comb_v, [rr, idx]
                    )

            pltpu.sync_copy(out_v, out_hbm.at[pl.ds(row0, CHUNK), :])
            return carry

        lax.fori_loop(0, n_chunks, chunk_body, 0)

    return k(positions, comb)


def kernel(positions, amplitudes, hermite_basis):
    batch, seq_len = positions.shape
    comb = _make_comb(amplitudes, hermite_basis)
    return _sc_gather(positions, comb, batch, seq_len)


# speedup vs baseline: 130.6473x; 1.2123x over previous
import functools

import jax
import jax.numpy as jnp
from jax import lax
from jax.experimental import pallas as pl
from jax.experimental.pallas import tpu as pltpu
from jax.experimental.pallas import tpu_sc as plsc

MAXN = 8
RES = 256
NC = 2
NS = 16
NW = NC * NS
LANES = 16
CHUNK = 32


def _comb_body(amp_ref, basis_ref, comb_ref):
    comb_ref[...] = jnp.dot(
        amp_ref[...], basis_ref[...], preferred_element_type=jnp.float32
    )


def _make_comb(amplitudes, basis):
    batch = amplitudes.shape[0]
    blk = 2048
    return pl.pallas_call(
        _comb_body,
        grid=(batch // blk,),
        in_specs=[
            pl.BlockSpec((blk, MAXN), lambda i: (i, 0)),
            pl.BlockSpec((MAXN, RES), lambda i: (0, 0)),
        ],
        out_specs=pl.BlockSpec((blk, RES), lambda i: (i, 0)),
        out_shape=jax.ShapeDtypeStruct((batch, RES), jnp.float32),
    )(amplitudes, basis)


def _sc_gather(positions, comb, batch, seq_len):
    rows_per_w = batch // NW
    n_chunks = rows_per_w // CHUNK
    offs = [o * LANES for o in range(seq_len // LANES)]
    if seq_len % LANES:
        offs.append(seq_len - LANES)
    mesh = plsc.VectorSubcoreMesh(core_axis_name="c", subcore_axis_name="s")

    @functools.partial(
        pl.kernel,
        out_type=jax.ShapeDtypeStruct((batch, seq_len), jnp.float32),
        mesh=mesh,
        scratch_types=[
            pltpu.VMEM((2, CHUNK, seq_len), jnp.float32),
            pltpu.VMEM((2, CHUNK, RES), jnp.float32),
            pltpu.VMEM((2, CHUNK, seq_len), jnp.float32),
            pltpu.SemaphoreType.DMA((2,)),
            pltpu.SemaphoreType.DMA((2,)),
        ],
        compiler_params=pltpu.CompilerParams(needs_layout_passes=False),
    )
    def k(pos_hbm, comb_hbm, out_hbm, pos_v, comb_v, out_v, sem_in, sem_out):
        wid = lax.axis_index("s") * NC + lax.axis_index("c")
        base = wid * rows_per_w

        def in_copies(kk, b):
            row0 = base + kk * CHUNK
            return (
                pltpu.make_async_copy(
                    pos_hbm.at[pl.ds(row0, CHUNK), :], pos_v.at[b], sem_in.at[b]
                ),
                pltpu.make_async_copy(
                    comb_hbm.at[pl.ds(row0, CHUNK), :], comb_v.at[b], sem_in.at[b]
                ),
            )

        def out_copy(kk, b):
            row0 = base + kk * CHUNK
            return pltpu.make_async_copy(
                out_v.at[b], out_hbm.at[pl.ds(row0, CHUNK), :], sem_out.at[b]
            )

        for b in range(2):
            for cp in in_copies(b, b):
                cp.start()

        def outer(i, carry):
            for b in range(2):
                kk = 2 * i + b
                for cp in in_copies(kk, b):
                    cp.wait()

                @pl.when(kk >= 2)
                def _():
                    out_copy(kk, b).wait()

                pos_b = pos_v.at[b]
                comb_b = comb_v.at[b]
                out_b = out_v.at[b]

                @plsc.parallel_loop(0, CHUNK, unroll=4)
                def row_body(r):
                    rr = jnp.full((LANES,), r, dtype=jnp.int32)
                    for off in offs:
                        p = pos_b[r, pl.ds(off, LANES)]
                        t = (p + 1.0) * 0.5 * 255.0
                        idx = jnp.clip(t.astype(jnp.int32), 0, 255)
                        out_b[r, pl.ds(off, LANES)] = plsc.load_gather(
                            comb_b, [rr, idx]
                        )

                out_copy(kk, b).start()

                @pl.when(kk + 2 < n_chunks)
                def _():
                    for cp in in_copies(kk + 2, b):
                        cp.start()

            return carry

        lax.fori_loop(0, n_chunks // 2, outer, 0)
        for b in range(2):
            out_copy(n_chunks - 2 + b, b).wait()

    return k(positions, comb)


def kernel(positions, amplitudes, hermite_basis):
    batch, seq_len = positions.shape
    comb = _make_comb(amplitudes, hermite_basis)
    return _sc_gather(positions, comb, batch, seq_len)
